# baseline (speedup 1.0000x reference)
import functools

import jax
import jax.numpy as jnp
from jax import lax
from jax.experimental import pallas as pl
from jax.experimental.pallas import tpu as pltpu
from jax.experimental.pallas import tpu_sc as plsc

N_TOK = 8192
D_IN = 256
D_HID = 1024
NE = 16
G0 = 4
G1 = 4
T = 256
TILES = 48
PAD_N = TILES * T

NC = 2
NS = 16
L = 16
NW = NC * NS
CHUNK = N_TOK // NW
NG = CHUNK // L


def _wid():
    return lax.axis_index("s") * NC + lax.axis_index("c")


def _sc_count_body(samples_hbm, counts_hbm, evs_hbm, rs_hbm,
                   cols_v, ev_v, r_v, cnt_v):
    wid = _wid()
    base = wid * CHUNK
    pltpu.sync_copy(samples_hbm.at[pl.ds(base, CHUNK), pl.ds(0, 128)], cols_v)
    lanes = lax.iota(jnp.int32, L)
    zero16 = jnp.zeros((L,), jnp.int32)
    one16 = jnp.ones((L,), jnp.int32)

    def group(g, cnt):
        rows = lanes + g * L
        x = plsc.load_gather(cols_v, [rows, zero16])
        y = plsc.load_gather(cols_v, [rows, one16])
        i0 = jnp.clip((x * G0).astype(jnp.int32), 0, G0 - 1)
        j0 = jnp.clip((y * G1).astype(jnp.int32), 0, G1 - 1)
        ev = j0 * G0 + i0
        r = jnp.zeros((L,), jnp.int32)
        for e in range(NE):
            m = ev == e
            mi = m.astype(jnp.int32)
            c = plsc.cumsum(mi)
            ce = jnp.sum(jnp.where(lanes == e, cnt, 0))
            r = jnp.where(m, ce + c - 1, r)
            cnt = cnt + jnp.where(lanes == e, jnp.sum(mi), 0)
        ev_v[pl.ds(g * L, L)] = ev
        r_v[pl.ds(g * L, L)] = r
        return cnt

    cnt = lax.fori_loop(0, NG, group, jnp.zeros((L,), jnp.int32))
    cnt_v[...] = cnt
    pltpu.sync_copy(cnt_v, counts_hbm.at[pl.ds(wid * L, L)])
    pltpu.sync_copy(ev_v, evs_hbm.at[wid])
    pltpu.sync_copy(r_v, rs_hbm.at[wid])


def _sc_scatter_body(samples_hbm, counts_hbm, evs_hbm, rs_hbm,
                     slots_hbm, te_hbm, tv_hbm, nu_hbm, xs_hbm,
                     counts_v, ev_v, r_v, slot_v, rows_v, te_v, tv_v, nu_v,
                     rsem, ssem):
    wid = _wid()
    base = wid * CHUNK
    rows_cp0 = pltpu.async_copy(
        samples_hbm.at[pl.ds(base, 128)], rows_v.at[pl.ds(0, 128)], rsem)
    rows_cp1 = pltpu.async_copy(
        samples_hbm.at[pl.ds(base + 128, 128)], rows_v.at[pl.ds(128, 128)],
        rsem)
    pltpu.sync_copy(counts_hbm, counts_v)
    pltpu.sync_copy(evs_hbm.at[wid], ev_v)
    pltpu.sync_copy(rs_hbm.at[wid], r_v)
    lanes = lax.iota(jnp.int32, L)

    tot = jnp.zeros((L,), jnp.int32)
    mybase = jnp.zeros((L,), jnp.int32)
    for w in range(NW):
        row = counts_v[pl.ds(w * L, L)]
        tot = tot + row
        mybase = mybase + jnp.where(w < wid, row, 0)
    tiles_e = (tot + (T - 1)) >> 8
    csum_t = plsc.cumsum(tiles_e)
    pad_off = (csum_t - tiles_e) * T
    base_e = pad_off + mybase

    copies = []
    for g in range(NG):
        if g == 0:
            rows_cp0.wait()
        elif g == NG // 2:
            rows_cp1.wait()
        ev = ev_v[pl.ds(g * L, L)]
        r = r_v[pl.ds(g * L, L)]
        slot = jnp.take_along_axis(base_e, ev, axis=0) + r
        slot_v[pl.ds(g * L, L)] = slot
        copies.append(pltpu.async_copy(
            rows_v.at[pl.ds(g * L, L)], xs_hbm.at[slot], ssem))

    @pl.when(wid == 0)
    def _():
        e_last = jnp.max(jnp.where(tiles_e > 0, lanes, 0))
        n_used = jnp.sum(jnp.where(lanes == NE - 1, csum_t, 0))
        for tg in range(TILES // L):
            tv = lanes + tg * L
            te = jnp.zeros((L,), jnp.int32)
            for e in range(NE):
                ce = jnp.sum(jnp.where(lanes == e, csum_t, 0))
                te = te + (tv >= ce).astype(jnp.int32)
            valid = (tv < n_used).astype(jnp.int32)
            te = jnp.where(valid == 1, jnp.clip(te, 0, NE - 1), e_last)
            te_v[pl.ds(tg * L, L)] = te
            tv_v[pl.ds(tg * L, L)] = valid
        nu_v[...] = jnp.zeros((L,), jnp.int32) + n_used
        pltpu.sync_copy(te_v, te_hbm)
        pltpu.sync_copy(tv_v, tv_hbm)
        pltpu.sync_copy(nu_v, nu_hbm)

    pltpu.sync_copy(slot_v.at[pl.ds(0, 128)], slots_hbm.at[wid, 0])
    pltpu.sync_copy(slot_v.at[pl.ds(128, 128)], slots_hbm.at[wid, 1])
    for cp in copies:
        cp.wait()


def _sc_gather_body(ys_hbm, slots_hbm, out_hbm, slots_v, buf0, buf1,
                    sem0, sem1):
    wid = _wid()
    base = wid * CHUNK
    pltpu.sync_copy(slots_hbm.at[wid], slots_v)
    cp0 = pltpu.async_copy(ys_hbm.at[slots_v.at[0]], buf0, sem0)
    cp1 = pltpu.async_copy(ys_hbm.at[slots_v.at[1]], buf1, sem1)
    cp0.wait()
    pltpu.sync_copy(buf0, out_hbm.at[pl.ds(base, 128)])
    cp1.wait()
    pltpu.sync_copy(buf1, out_hbm.at[pl.ds(base + 128, 128)])


def _mlp_body(te_ref, tv_ref, nu_ref, xs_ref, w1_ref, b1_ref, w2_ref, b2_ref,
              out_ref):
    i = pl.program_id(0)

    @pl.when(tv_ref[i] == 1)
    def _():
        e = te_ref[i]
        x = xs_ref[...]
        h = jnp.dot(x, w1_ref[e], preferred_element_type=jnp.float32)
        h = jnp.maximum(h + b1_ref[e], 0.0)
        y = jnp.dot(h, w2_ref[e], preferred_element_type=jnp.float32)
        out_ref[...] = y + b2_ref[e]


def _mlp(tile_e, tile_v, n_used, xs, W1, b1, W2, b2):
    grid_spec = pltpu.PrefetchScalarGridSpec(
        num_scalar_prefetch=3,
        grid=(TILES,),
        in_specs=[
            pl.BlockSpec((T, D_IN),
                         lambda i, te, tv, nu: (jnp.minimum(i, nu[0] - 1), 0)),
            pl.BlockSpec((NE, D_IN, D_HID), lambda i, te, tv, nu: (0, 0, 0)),
            pl.BlockSpec((NE, 1, D_HID), lambda i, te, tv, nu: (0, 0, 0)),
            pl.BlockSpec((NE, D_HID, D_IN), lambda i, te, tv, nu: (0, 0, 0)),
            pl.BlockSpec((NE, 1, D_IN), lambda i, te, tv, nu: (0, 0, 0)),
        ],
        out_specs=pl.BlockSpec(
            (T, D_IN), lambda i, te, tv, nu: (jnp.minimum(i, nu[0] - 1), 0)),
    )
    return pl.pallas_call(
        _mlp_body,
        grid_spec=grid_spec,
        out_shape=jax.ShapeDtypeStruct((PAD_N, D_IN), jnp.float32),
        compiler_params=pltpu.CompilerParams(dimension_semantics=("arbitrary",)),
    )(tile_e, tile_v, n_used, xs, W1, b1.reshape(NE, 1, D_HID), W2,
      b2.reshape(NE, 1, D_IN))


@functools.lru_cache(maxsize=1)
def _build_sc_kernels():
    mesh = plsc.VectorSubcoreMesh(core_axis_name="c", subcore_axis_name="s",
                                  num_cores=NC)
    sc_params = pltpu.CompilerParams(needs_layout_passes=False)
    sc_count = pl.kernel(
        _sc_count_body,
        mesh=mesh,
        compiler_params=sc_params,
        out_type=[
            jax.ShapeDtypeStruct((NW * L,), jnp.int32),
            jax.ShapeDtypeStruct((NW, CHUNK), jnp.int32),
            jax.ShapeDtypeStruct((NW, CHUNK), jnp.int32),
        ],
        scratch_types=[
            pltpu.VMEM((CHUNK, 128), jnp.float32),
            pltpu.VMEM((CHUNK,), jnp.int32),
            pltpu.VMEM((CHUNK,), jnp.int32),
            pltpu.VMEM((L,), jnp.int32),
        ],
    )
    sc_scatter = pl.kernel(
        _sc_scatter_body,
        mesh=mesh,
        compiler_params=sc_params,
        out_type=[
            jax.ShapeDtypeStruct((NW, 2, 128), jnp.int32),
            jax.ShapeDtypeStruct((TILES,), jnp.int32),
            jax.ShapeDtypeStruct((TILES,), jnp.int32),
            jax.ShapeDtypeStruct((L,), jnp.int32),
            jax.ShapeDtypeStruct((PAD_N, D_IN), jnp.float32),
        ],
        scratch_types=[
            pltpu.VMEM((NW * L,), jnp.int32),
            pltpu.VMEM((CHUNK,), jnp.int32),
            pltpu.VMEM((CHUNK,), jnp.int32),
            pltpu.VMEM((CHUNK,), jnp.int32),
            pltpu.VMEM((CHUNK, D_IN), jnp.float32),
            pltpu.VMEM((TILES,), jnp.int32),
            pltpu.VMEM((TILES,), jnp.int32),
            pltpu.VMEM((L,), jnp.int32),
            pltpu.SemaphoreType.DMA,
            pltpu.SemaphoreType.DMA,
        ],
    )
    sc_gather = pl.kernel(
        _sc_gather_body,
        mesh=mesh,
        compiler_params=sc_params,
        out_type=jax.ShapeDtypeStruct((N_TOK, D_IN), jnp.float32),
        scratch_types=[
            pltpu.VMEM((2, 128), jnp.int32),
            pltpu.VMEM((128, D_IN), jnp.float32),
            pltpu.VMEM((128, D_IN), jnp.float32),
            pltpu.SemaphoreType.DMA,
            pltpu.SemaphoreType.DMA,
        ],
    )
    return sc_count, sc_scatter, sc_gather


def kernel(samples, W1, b1, W2, b2):
    sc_count, sc_scatter, sc_gather = _build_sc_kernels()
    counts, evs, rs = sc_count(samples)
    slots, tile_e, tile_v, n_used, xs = sc_scatter(samples, counts, evs, rs)
    ys = _mlp(tile_e, tile_v, n_used, xs, W1, b1, W2, b2)
    return sc_gather(ys, slots)


# speedup vs baseline: 1.0161x; 1.0161x over previous
import functools

import jax
import jax.numpy as jnp
from jax import lax
from jax.experimental import pallas as pl
from jax.experimental.pallas import tpu as pltpu
from jax.experimental.pallas import tpu_sc as plsc

N_TOK = 8192
D_IN = 256
D_HID = 1024
NE = 16
G0 = 4
G1 = 4
T = 256
TILES = 48
PAD_N = TILES * T

NC = 2
NS = 16
L = 16
NW = NC * NS
CHUNK = N_TOK // NW
NG = CHUNK // L


def _wid():
    return lax.axis_index("s") * NC + lax.axis_index("c")


def _sc_count_body(samples_hbm, counts_hbm, evs_hbm, rs_hbm,
                   cols_v, ev_v, r_v, cnt_v):
    wid = _wid()
    base = wid * CHUNK
    pltpu.sync_copy(samples_hbm.at[pl.ds(base, CHUNK), pl.ds(0, 128)], cols_v)
    lanes = lax.iota(jnp.int32, L)
    zero16 = jnp.zeros((L,), jnp.int32)
    one16 = jnp.ones((L,), jnp.int32)

    def group(g, cnt):
        rows = lanes + g * L
        x = plsc.load_gather(cols_v, [rows, zero16])
        y = plsc.load_gather(cols_v, [rows, one16])
        i0 = jnp.clip((x * G0).astype(jnp.int32), 0, G0 - 1)
        j0 = jnp.clip((y * G1).astype(jnp.int32), 0, G1 - 1)
        ev = j0 * G0 + i0
        r = jnp.zeros((L,), jnp.int32)
        for e in range(NE):
            m = ev == e
            mi = m.astype(jnp.int32)
            c = plsc.cumsum(mi)
            ce = jnp.sum(jnp.where(lanes == e, cnt, 0))
            r = jnp.where(m, ce + c - 1, r)
            cnt = cnt + jnp.where(lanes == e, jnp.sum(mi), 0)
        ev_v[pl.ds(g * L, L)] = ev
        r_v[pl.ds(g * L, L)] = r
        return cnt

    cnt = lax.fori_loop(0, NG, group, jnp.zeros((L,), jnp.int32))
    cnt_v[...] = cnt
    pltpu.sync_copy(cnt_v, counts_hbm.at[pl.ds(wid * L, L)])
    pltpu.sync_copy(ev_v, evs_hbm.at[wid])
    pltpu.sync_copy(r_v, rs_hbm.at[wid])


def _sc_scatter_body(samples_hbm, counts_hbm, evs_hbm, rs_hbm,
                     slots_hbm, te_hbm, tv_hbm, nu_hbm, xs_hbm,
                     counts_v, ev_v, r_v, slot_v, rows_v, te_v, tv_v, nu_v,
                     rsem, ssem):
    wid = _wid()
    base = wid * CHUNK
    rows_cp = pltpu.async_copy(samples_hbm.at[pl.ds(base, CHUNK)], rows_v, rsem)
    pltpu.sync_copy(counts_hbm, counts_v)
    pltpu.sync_copy(evs_hbm.at[wid], ev_v)
    pltpu.sync_copy(rs_hbm.at[wid], r_v)
    lanes = lax.iota(jnp.int32, L)

    tot = jnp.zeros((L,), jnp.int32)
    mybase = jnp.zeros((L,), jnp.int32)
    for w in range(NW):
        row = counts_v[pl.ds(w * L, L)]
        tot = tot + row
        mybase = mybase + jnp.where(w < wid, row, 0)
    tiles_e = (tot + (T - 1)) >> 8
    csum_t = plsc.cumsum(tiles_e)
    pad_off = (csum_t - tiles_e) * T
    base_e = pad_off + mybase

    rows_cp.wait()
    copies = []
    for g in range(NG):
        ev = ev_v[pl.ds(g * L, L)]
        r = r_v[pl.ds(g * L, L)]
        slot = jnp.take_along_axis(base_e, ev, axis=0) + r
        slot_v[pl.ds(g * L, L)] = slot
        copies.append(pltpu.async_copy(
            rows_v.at[pl.ds(g * L, L)], xs_hbm.at[slot], ssem))

    @pl.when(wid == 0)
    def _():
        e_last = jnp.max(jnp.where(tiles_e > 0, lanes, 0))
        n_used = jnp.sum(jnp.where(lanes == NE - 1, csum_t, 0))
        for tg in range(TILES // L):
            tv = lanes + tg * L
            te = jnp.zeros((L,), jnp.int32)
            for e in range(NE):
                ce = jnp.sum(jnp.where(lanes == e, csum_t, 0))
                te = te + (tv >= ce).astype(jnp.int32)
            valid = (tv < n_used).astype(jnp.int32)
            te = jnp.where(valid == 1, jnp.clip(te, 0, NE - 1), e_last)
            te_v[pl.ds(tg * L, L)] = te
            tv_v[pl.ds(tg * L, L)] = valid
        nu_v[...] = jnp.zeros((L,), jnp.int32) + n_used
        pltpu.sync_copy(te_v, te_hbm)
        pltpu.sync_copy(tv_v, tv_hbm)
        pltpu.sync_copy(nu_v, nu_hbm)

    pltpu.sync_copy(slot_v.at[pl.ds(0, 128)], slots_hbm.at[wid, 0])
    pltpu.sync_copy(slot_v.at[pl.ds(128, 128)], slots_hbm.at[wid, 1])
    for cp in copies:
        cp.wait()


def _sc_gather_body(ys_hbm, slots_hbm, out_hbm, slots_v, buf0, buf1,
                    sem0, sem1):
    wid = _wid()
    base = wid * CHUNK
    pltpu.sync_copy(slots_hbm.at[wid], slots_v)
    cp0 = pltpu.async_copy(ys_hbm.at[slots_v.at[0]], buf0, sem0)
    cp1 = pltpu.async_copy(ys_hbm.at[slots_v.at[1]], buf1, sem1)
    cp0.wait()
    pltpu.sync_copy(buf0, out_hbm.at[pl.ds(base, 128)])
    cp1.wait()
    pltpu.sync_copy(buf1, out_hbm.at[pl.ds(base + 128, 128)])


def _mlp_body(te_ref, tv_ref, nu_ref, xs_ref, w1_ref, b1_ref, w2_ref, b2_ref,
              out_ref):
    i = pl.program_id(0)

    @pl.when(tv_ref[i] == 1)
    def _():
        e = te_ref[i]
        x = xs_ref[...]
        h = jnp.dot(x, w1_ref[e], preferred_element_type=jnp.float32)
        h = jnp.maximum(h + b1_ref[e], 0.0)
        y = jnp.dot(h, w2_ref[e], preferred_element_type=jnp.float32)
        out_ref[...] = y + b2_ref[e]


def _mlp(tile_e, tile_v, n_used, xs, W1, b1, W2, b2):
    grid_spec = pltpu.PrefetchScalarGridSpec(
        num_scalar_prefetch=3,
        grid=(TILES,),
        in_specs=[
            pl.BlockSpec((T, D_IN),
                         lambda i, te, tv, nu: (jnp.minimum(i, nu[0] - 1), 0)),
            pl.BlockSpec((NE, D_IN, D_HID), lambda i, te, tv, nu: (0, 0, 0)),
            pl.BlockSpec((NE, 1, D_HID), lambda i, te, tv, nu: (0, 0, 0)),
            pl.BlockSpec((NE, D_HID, D_IN), lambda i, te, tv, nu: (0, 0, 0)),
            pl.BlockSpec((NE, 1, D_IN), lambda i, te, tv, nu: (0, 0, 0)),
        ],
        out_specs=pl.BlockSpec(
            (T, D_IN), lambda i, te, tv, nu: (jnp.minimum(i, nu[0] - 1), 0)),
    )
    return pl.pallas_call(
        _mlp_body,
        grid_spec=grid_spec,
        out_shape=jax.ShapeDtypeStruct((PAD_N, D_IN), jnp.float32),
        compiler_params=pltpu.CompilerParams(dimension_semantics=("arbitrary",)),
    )(tile_e, tile_v, n_used, xs, W1, b1.reshape(NE, 1, D_HID), W2,
      b2.reshape(NE, 1, D_IN))


@functools.lru_cache(maxsize=1)
def _build_sc_kernels():
    mesh = plsc.VectorSubcoreMesh(core_axis_name="c", subcore_axis_name="s",
                                  num_cores=NC)
    sc_params = pltpu.CompilerParams(needs_layout_passes=False)
    sc_count = pl.kernel(
        _sc_count_body,
        mesh=mesh,
        compiler_params=sc_params,
        out_type=[
            jax.ShapeDtypeStruct((NW * L,), jnp.int32),
            jax.ShapeDtypeStruct((NW, CHUNK), jnp.int32),
            jax.ShapeDtypeStruct((NW, CHUNK), jnp.int32),
        ],
        scratch_types=[
            pltpu.VMEM((CHUNK, 128), jnp.float32),
            pltpu.VMEM((CHUNK,), jnp.int32),
            pltpu.VMEM((CHUNK,), jnp.int32),
            pltpu.VMEM((L,), jnp.int32),
        ],
    )
    sc_scatter = pl.kernel(
        _sc_scatter_body,
        mesh=mesh,
        compiler_params=sc_params,
        out_type=[
            jax.ShapeDtypeStruct((NW, 2, 128), jnp.int32),
            jax.ShapeDtypeStruct((TILES,), jnp.int32),
            jax.ShapeDtypeStruct((TILES,), jnp.int32),
            jax.ShapeDtypeStruct((L,), jnp.int32),
            jax.ShapeDtypeStruct((PAD_N, D_IN), jnp.float32),
        ],
        scratch_types=[
            pltpu.VMEM((NW * L,), jnp.int32),
            pltpu.VMEM((CHUNK,), jnp.int32),
            pltpu.VMEM((CHUNK,), jnp.int32),
            pltpu.VMEM((CHUNK,), jnp.int32),
            pltpu.VMEM((CHUNK, D_IN), jnp.float32),
            pltpu.VMEM((TILES,), jnp.int32),
            pltpu.VMEM((TILES,), jnp.int32),
            pltpu.VMEM((L,), jnp.int32),
            pltpu.SemaphoreType.DMA,
            pltpu.SemaphoreType.DMA,
        ],
    )
    sc_gather = pl.kernel(
        _sc_gather_body,
        mesh=mesh,
        compiler_params=sc_params,
        out_type=jax.ShapeDtypeStruct((N_TOK, D_IN), jnp.float32),
        scratch_types=[
            pltpu.VMEM((2, 128), jnp.int32),
            pltpu.VMEM((128, D_IN), jnp.float32),
            pltpu.VMEM((128, D_IN), jnp.float32),
            pltpu.SemaphoreType.DMA,
            pltpu.SemaphoreType.DMA,
        ],
    )
    return sc_count, sc_scatter, sc_gather


def kernel(samples, W1, b1, W2, b2):
    sc_count, sc_scatter, sc_gather = _build_sc_kernels()
    counts, evs, rs = sc_count(samples)
    slots, tile_e, tile_v, n_used, xs = sc_scatter(samples, counts, evs, rs)
    ys = _mlp(tile_e, tile_v, n_used, xs, W1, b1, W2, b2)
    return sc_gather(ys, slots)
